# Initial kernel scaffold; baseline (speedup 1.0000x reference)
import jax
import jax.numpy as jnp
from jax.experimental import pallas as pl


def kernel(h, edge_index, edge_attr, W_e1, b_e1, g_e, bb_e, W_e2, b_e2, W_n1, b_n1, g_n, bb_n, W_n2, b_n2):
    raise NotImplementedError("write your pallas kernel here")


# speedup vs baseline: 2.1845x; 2.1845x over previous
import functools

import jax
import jax.numpy as jnp
from jax import lax
from jax.experimental import pallas as pl
from jax.experimental.pallas import tpu as pltpu
from jax.experimental.pallas import tpu_sc as plsc

_NC = 2
_NS = 16
_NW = _NC * _NS


def _sc_gather(h, row, col):
    N, D = h.shape
    E = row.shape[0]
    per_w = E // _NW
    C = 80
    n_ch = per_w // C
    assert per_w * _NW == E and n_ch * C == per_w

    mesh = plsc.VectorSubcoreMesh(core_axis_name="c", subcore_axis_name="s", num_cores=_NC, num_subcores=_NS)

    @functools.partial(
        pl.kernel,
        out_type=(jax.ShapeDtypeStruct((E, D), jnp.float32),
                  jax.ShapeDtypeStruct((E, D), jnp.float32)),
        mesh=mesh,
        scratch_types=[
            pltpu.VMEM((C,), jnp.int32),
            pltpu.VMEM((C,), jnp.int32),
            pltpu.VMEM((C, D), jnp.float32),
            pltpu.VMEM((C, D), jnp.float32),
            pltpu.SemaphoreType.DMA,
            pltpu.SemaphoreType.DMA,
        ],
    )
    def k(h_hbm, row_hbm, col_hbm, s_out, t_out, ridx, cidx, sbuf, tbuf,
          sem_a, sem_b):
        c = lax.axis_index("c")
        s = lax.axis_index("s")
        w = s * _NC + c
        base = w * per_w

        def body(i, carry):
            off = base + i * C
            pltpu.sync_copy(row_hbm.at[pl.ds(off, C)], ridx)
            pltpu.sync_copy(col_hbm.at[pl.ds(off, C)], cidx)
            ca = pltpu.async_copy(h_hbm.at[ridx], sbuf, sem_a)
            cb = pltpu.async_copy(h_hbm.at[cidx], tbuf, sem_b)
            ca.wait()
            cb.wait()
            pltpu.sync_copy(sbuf, s_out.at[pl.ds(off, C)])
            pltpu.sync_copy(tbuf, t_out.at[pl.ds(off, C)])
            return carry

        lax.fori_loop(0, n_ch, body, 0)

    return k(h, row, col)


def _sc_scatter(mij, row, N):
    E, H = mij.shape
    per_w = E // _NS
    C = 80
    n_ch = per_w // C
    rpt = -(-N // (8 * _NS * _NC)) * 8
    rpc = rpt * _NS
    Np = rpc * _NC
    lanes = C // 16
    assert n_ch * C == per_w

    mesh = plsc.VectorSubcoreMesh(core_axis_name="c", subcore_axis_name="s",
                                  num_cores=_NC, num_subcores=_NS)

    @functools.partial(
        pl.kernel,
        out_type=jax.ShapeDtypeStruct((Np, H), jnp.float32),
        mesh=mesh,
        scratch_types=[
            pltpu.VMEM((C,), jnp.int32),
            pltpu.VMEM((C,), jnp.int32),
            pltpu.VMEM((C, H), jnp.float32),
            pltpu.VMEM((rpt, H), jnp.float32),
            pltpu.VMEM_SHARED((rpc + 8, H), jnp.float32),
            pltpu.SemaphoreType.DMA,
        ],
    )
    def k(mij_hbm, row_hbm, out_hbm, idx, idx2, buf, zbuf, agg_sh, sem):
        c = lax.axis_index("c")
        s = lax.axis_index("s")
        hl = H // 16
        zv = jnp.zeros((16,), jnp.float32)

        def zrow(i, carry):
            zbuf[i // hl, pl.ds((i % hl) * 16, 16)] = zv
            return carry

        lax.fori_loop(0, rpt * hl, zrow, 0)
        pltpu.sync_copy(zbuf, agg_sh.at[pl.ds(s * rpt, rpt)])

        @pl.when(s == 0)
        def _():
            pltpu.sync_copy(zbuf.at[pl.ds(0, 8)], agg_sh.at[pl.ds(rpc, 8)])

        plsc.subcore_barrier()

        base_row = c * rpc
        base = s * per_w

        def body(i, carry):
            off = base + i * C
            pltpu.sync_copy(row_hbm.at[pl.ds(off, C)], idx)
            pltpu.sync_copy(mij_hbm.at[pl.ds(off, C)], buf)
            for j in range(lanes):
                v = idx[pl.ds(j * 16, 16)] - base_row
                ok = (v >= 0) & (v < rpc)
                idx2[pl.ds(j * 16, 16)] = jnp.where(ok, v, rpc)
            pltpu.sync_copy(buf, agg_sh.at[idx2], add=True)
            return carry

        lax.fori_loop(0, n_ch, body, 0)
        plsc.subcore_barrier()
        pltpu.sync_copy(agg_sh.at[pl.ds(s * rpt, rpt)],
                        out_hbm.at[pl.ds(base_row + s * rpt, rpt)])

    return k(mij, row)


def _silu(x):
    return x * jax.nn.sigmoid(x)


def _tc_edge(S, T, ea, w1a, w1b, w1c, b1, g, bb, w2, b2):
    E, D = S.shape
    DE = ea.shape[1]
    H = w2.shape[1]
    BE = 2560 if E % 2560 == 0 else E
    grid = E // BE
    assert grid * BE == E

    def body(s_ref, t_ref, e_ref, w1a_r, w1b_r, w1c_r, b1_r, g_r, bb_r,
             w2_r, b2_r, out_ref):
        x = (jnp.dot(s_ref[...], w1a_r[...], preferred_element_type=jnp.float32)
             + jnp.dot(t_ref[...], w1b_r[...], preferred_element_type=jnp.float32)
             + jnp.dot(e_ref[...], w1c_r[...], preferred_element_type=jnp.float32)
             + b1_r[...])
        mu = jnp.mean(x, axis=-1, keepdims=True)
        var = jnp.mean((x - mu) ** 2, axis=-1, keepdims=True)
        xn = (x - mu) / jnp.sqrt(var + 1e-5) * g_r[...] + bb_r[...]
        m = _silu(xn)
        y = jnp.dot(m, w2_r[...], preferred_element_type=jnp.float32) + b2_r[...]
        out_ref[...] = _silu(y)

    full = lambda r, c: pl.BlockSpec((r, c), lambda i: (0, 0))
    return pl.pallas_call(
        body,
        grid=(grid,),
        in_specs=[
            pl.BlockSpec((BE, D), lambda i: (i, 0)),
            pl.BlockSpec((BE, D), lambda i: (i, 0)),
            pl.BlockSpec((BE, DE), lambda i: (i, 0)),
            full(D, H), full(D, H), full(DE, H), full(1, H), full(1, H),
            full(1, H), full(H, H), full(1, H),
        ],
        out_specs=pl.BlockSpec((BE, H), lambda i: (i, 0)),
        out_shape=jax.ShapeDtypeStruct((E, H), jnp.float32),
    )(S, T, ea, w1a, w1b, w1c, b1.reshape(1, H), g.reshape(1, H),
      bb.reshape(1, H), w2, b2.reshape(1, H))


def _tc_node(h, p, wn1a, wn1b, bn1, gn, bbn, wn2, bn2):
    N, D = h.shape
    H = wn1a.shape[1]
    BN = 2000 if N % 2000 == 0 else N
    grid = N // BN
    assert grid * BN == N

    def body(h_ref, p_ref, wa_r, wb_r, b1_r, g_r, bb_r, w2_r, b2_r,
             out_ref):
        agg = p_ref[...] * jnp.float32(0.01)
        x = (jnp.dot(h_ref[...], wa_r[...], preferred_element_type=jnp.float32)
             + jnp.dot(agg, wb_r[...], preferred_element_type=jnp.float32)
             + b1_r[...])
        mu = jnp.mean(x, axis=-1, keepdims=True)
        var = jnp.mean((x - mu) ** 2, axis=-1, keepdims=True)
        xn = (x - mu) / jnp.sqrt(var + 1e-5) * g_r[...] + bb_r[...]
        nh = _silu(xn)
        y = jnp.dot(nh, w2_r[...], preferred_element_type=jnp.float32) + b2_r[...]
        out_ref[...] = h_ref[...] + y

    full = lambda r, c: pl.BlockSpec((r, c), lambda i: (0, 0))
    return pl.pallas_call(
        body,
        grid=(grid,),
        in_specs=[
            pl.BlockSpec((BN, D), lambda i: (i, 0)),
            pl.BlockSpec((BN, D), lambda i: (i, 0)),
            full(D, H), full(D, H), full(1, H), full(1, H), full(1, H),
            full(H, D), full(1, D),
        ],
        out_specs=pl.BlockSpec((BN, D), lambda i: (i, 0)),
        out_shape=jax.ShapeDtypeStruct((N, D), jnp.float32),
    )(h, p, wn1a, wn1b, bn1.reshape(1, H), gn.reshape(1, H),
      bbn.reshape(1, H), wn2, bn2.reshape(1, D))


def kernel(h, edge_index, edge_attr, W_e1, b_e1, g_e, bb_e, W_e2, b_e2,
           W_n1, b_n1, g_n, bb_n, W_n2, b_n2):
    N, D = h.shape
    row, col = edge_index[0], edge_index[1]
    S, T = _sc_gather(h, row, col)
    mij = _tc_edge(S, T, edge_attr, W_e1[:D], W_e1[D:2 * D], W_e1[2 * D:],
                   b_e1, g_e, bb_e, W_e2, b_e2)
    agg = _sc_scatter(mij, row, N)
    h_out = _tc_node(h, agg[:N], W_n1[:D], W_n1[D:], b_n1,
                     g_n, bb_n, W_n2, b_n2)
    return (h_out, mij)


# speedup vs baseline: 3.0941x; 1.4164x over previous
import functools

import jax
import jax.numpy as jnp
from jax import lax
from jax.experimental import pallas as pl
from jax.experimental.pallas import tpu as pltpu
from jax.experimental.pallas import tpu_sc as plsc

_NC = 2
_NS = 16
_NW = _NC * _NS


def _sc_gather(h, row, col):
    N, D = h.shape
    E = row.shape[0]
    per_w = E // _NW
    C = 80
    n_ch = per_w // C
    NBUF = 5
    rounds = n_ch // NBUF
    assert per_w * _NW == E and n_ch * C == per_w and rounds * NBUF == n_ch

    mesh = plsc.VectorSubcoreMesh(core_axis_name="c", subcore_axis_name="s",
                                  num_cores=_NC, num_subcores=_NS)

    @functools.partial(
        pl.kernel,
        out_type=(jax.ShapeDtypeStruct((E, D), jnp.float32),
                  jax.ShapeDtypeStruct((E, D), jnp.float32)),
        mesh=mesh,
        scratch_types=(
            [pltpu.VMEM((per_w,), jnp.int32), pltpu.VMEM((per_w,), jnp.int32)]
            + [pltpu.VMEM((C, D), jnp.float32)] * (2 * NBUF)
            + [pltpu.SemaphoreType.DMA] * (2 * NBUF)
        ),
    )
    def k(h_hbm, row_hbm, col_hbm, s_out, t_out, ridx, cidx, *rest):
        sbufs = rest[0:NBUF]
        tbufs = rest[NBUF:2 * NBUF]
        gsems = rest[2 * NBUF:3 * NBUF]
        wsems = rest[3 * NBUF:4 * NBUF]
        c = lax.axis_index("c")
        s = lax.axis_index("s")
        w = s * _NC + c
        base = w * per_w
        pltpu.sync_copy(row_hbm.at[pl.ds(base, per_w)], ridx)
        pltpu.sync_copy(col_hbm.at[pl.ds(base, per_w)], cidx)

        def fire(ch, b):
            pltpu.async_copy(h_hbm.at[ridx.at[pl.ds(ch * C, C)]], sbufs[b],
                             gsems[b])
            pltpu.async_copy(h_hbm.at[cidx.at[pl.ds(ch * C, C)]], tbufs[b],
                             gsems[b])

        for b in range(NBUF):
            fire(b, b)

        def round_(q, carry):
            for b in range(NBUF):
                ch = q * NBUF + b
                off = base + ch * C
                pltpu.make_async_copy(h_hbm.at[ridx.at[pl.ds(0, C)]],
                                      sbufs[b], gsems[b]).wait()
                pltpu.make_async_copy(h_hbm.at[cidx.at[pl.ds(0, C)]],
                                      tbufs[b], gsems[b]).wait()
                pltpu.async_copy(sbufs[b], s_out.at[pl.ds(off, C)], wsems[b])
                pltpu.async_copy(tbufs[b], t_out.at[pl.ds(off, C)], wsems[b])
            for b in range(NBUF):
                ch = q * NBUF + b
                pltpu.make_async_copy(sbufs[b], s_out.at[pl.ds(base, C)],
                                      wsems[b]).wait()
                pltpu.make_async_copy(tbufs[b], t_out.at[pl.ds(base, C)],
                                      wsems[b]).wait()

                @pl.when(q < rounds - 1)
                def _():
                    fire(ch + NBUF, b)

            return carry

        lax.fori_loop(0, rounds, round_, 0)

    return k(h, row, col)


def _sc_scatter(mij, row, N):
    E, H = mij.shape
    per_w = E // _NS
    C = 80
    n_ch = per_w // C
    NBUF = 5
    rounds = n_ch // NBUF
    rpt = -(-N // (8 * _NS * _NC)) * 8
    rpc = rpt * _NS
    Np = rpc * _NC
    lanes = C // 16
    assert n_ch * C == per_w and rounds * NBUF == n_ch and rpt % C == 0

    mesh = plsc.VectorSubcoreMesh(core_axis_name="c", subcore_axis_name="s",
                                  num_cores=_NC, num_subcores=_NS)

    @functools.partial(
        pl.kernel,
        out_type=jax.ShapeDtypeStruct((Np, H), jnp.float32),
        mesh=mesh,
        scratch_types=(
            [pltpu.VMEM((per_w,), jnp.int32)]
            + [pltpu.VMEM((C, H), jnp.float32)] * NBUF
            + [pltpu.VMEM((C,), jnp.int32)] * NBUF
            + [pltpu.VMEM_SHARED((rpc + 8, H), jnp.float32)]
            + [pltpu.SemaphoreType.DMA] * (2 * NBUF)
        ),
    )
    def k(mij_hbm, row_hbm, out_hbm, idx_all, *rest):
        bufs = rest[0:NBUF]
        idx2 = rest[NBUF:2 * NBUF]
        agg_sh = rest[2 * NBUF]
        lsems = rest[2 * NBUF + 1:3 * NBUF + 1]
        ssems = rest[3 * NBUF + 1:4 * NBUF + 1]
        c = lax.axis_index("c")
        s = lax.axis_index("s")
        hl = H // 16
        zv = jnp.zeros((16,), jnp.float32)

        def zrow(i, carry):
            bufs[0][i // hl, pl.ds((i % hl) * 16, 16)] = zv
            return carry

        lax.fori_loop(0, C * hl, zrow, 0)
        for t in range(rpt // C):
            pltpu.sync_copy(bufs[0], agg_sh.at[pl.ds(s * rpt + t * C, C)])

        @pl.when(s == 0)
        def _():
            pltpu.sync_copy(bufs[0].at[pl.ds(0, 8)], agg_sh.at[pl.ds(rpc, 8)])

        base_row = c * rpc
        base = s * per_w
        pltpu.sync_copy(row_hbm.at[pl.ds(base, per_w)], idx_all)
        plsc.subcore_barrier()

        def fire(ch, b):
            pltpu.async_copy(mij_hbm.at[pl.ds(base + ch * C, C)], bufs[b],
                             lsems[b])

        for b in range(NBUF):
            fire(b, b)

        def round_(q, carry):
            for b in range(NBUF):
                ch = q * NBUF + b
                pltpu.make_async_copy(mij_hbm.at[pl.ds(base, C)], bufs[b],
                                      lsems[b]).wait()
                for j in range(lanes):
                    v = idx_all[pl.ds(ch * C + j * 16, 16)] - base_row
                    ok = (v >= 0) & (v < rpc)
                    idx2[b][pl.ds(j * 16, 16)] = jnp.where(ok, v, rpc)
                pltpu.async_copy(bufs[b], agg_sh.at[idx2[b]], ssems[b],
                                 add=True)
            for b in range(NBUF):
                ch = q * NBUF + b
                pltpu.make_async_copy(bufs[b], agg_sh.at[idx2[b]],
                                      ssems[b]).wait()

                @pl.when(q < rounds - 1)
                def _():
                    fire(ch + NBUF, b)

            return carry

        lax.fori_loop(0, rounds, round_, 0)
        plsc.subcore_barrier()
        pltpu.sync_copy(agg_sh.at[pl.ds(s * rpt, rpt)],
                        out_hbm.at[pl.ds(base_row + s * rpt, rpt)])

    return k(mij, row)


def _silu(x):
    return x * jax.nn.sigmoid(x)


def _tc_edge(S, T, ea, w1a, w1b, w1c, b1, g, bb, w2, b2):
    E, D = S.shape
    DE = ea.shape[1]
    H = w2.shape[1]
    BE = 2560 if E % 2560 == 0 else E
    grid = E // BE
    assert grid * BE == E

    def body(s_ref, t_ref, e_ref, w1a_r, w1b_r, w1c_r, b1_r, g_r, bb_r,
             w2_r, b2_r, out_ref):
        x = (jnp.dot(s_ref[...], w1a_r[...], preferred_element_type=jnp.float32)
             + jnp.dot(t_ref[...], w1b_r[...], preferred_element_type=jnp.float32)
             + jnp.dot(e_ref[...], w1c_r[...], preferred_element_type=jnp.float32)
             + b1_r[...])
        mu = jnp.mean(x, axis=-1, keepdims=True)
        var = jnp.mean((x - mu) ** 2, axis=-1, keepdims=True)
        xn = (x - mu) / jnp.sqrt(var + 1e-5) * g_r[...] + bb_r[...]
        m = _silu(xn)
        y = jnp.dot(m, w2_r[...], preferred_element_type=jnp.float32) + b2_r[...]
        out_ref[...] = _silu(y)

    full = lambda r, c: pl.BlockSpec((r, c), lambda i: (0, 0))
    return pl.pallas_call(
        body,
        grid=(grid,),
        in_specs=[
            pl.BlockSpec((BE, D), lambda i: (i, 0)),
            pl.BlockSpec((BE, D), lambda i: (i, 0)),
            pl.BlockSpec((BE, DE), lambda i: (i, 0)),
            full(D, H), full(D, H), full(DE, H), full(1, H), full(1, H),
            full(1, H), full(H, H), full(1, H),
        ],
        out_specs=pl.BlockSpec((BE, H), lambda i: (i, 0)),
        out_shape=jax.ShapeDtypeStruct((E, H), jnp.float32),
    )(S, T, ea, w1a, w1b, w1c, b1.reshape(1, H), g.reshape(1, H),
      bb.reshape(1, H), w2, b2.reshape(1, H))


def _tc_node(h, p, wn1a, wn1b, bn1, gn, bbn, wn2, bn2):
    N, D = h.shape
    H = wn1a.shape[1]
    BN = 2000 if N % 2000 == 0 else N
    grid = N // BN
    assert grid * BN == N

    def body(h_ref, p_ref, wa_r, wb_r, b1_r, g_r, bb_r, w2_r, b2_r,
             out_ref):
        agg = p_ref[...] * jnp.float32(0.01)
        x = (jnp.dot(h_ref[...], wa_r[...], preferred_element_type=jnp.float32)
             + jnp.dot(agg, wb_r[...], preferred_element_type=jnp.float32)
             + b1_r[...])
        mu = jnp.mean(x, axis=-1, keepdims=True)
        var = jnp.mean((x - mu) ** 2, axis=-1, keepdims=True)
        xn = (x - mu) / jnp.sqrt(var + 1e-5) * g_r[...] + bb_r[...]
        nh = _silu(xn)
        y = jnp.dot(nh, w2_r[...], preferred_element_type=jnp.float32) + b2_r[...]
        out_ref[...] = h_ref[...] + y

    full = lambda r, c: pl.BlockSpec((r, c), lambda i: (0, 0))
    return pl.pallas_call(
        body,
        grid=(grid,),
        in_specs=[
            pl.BlockSpec((BN, D), lambda i: (i, 0)),
            pl.BlockSpec((BN, D), lambda i: (i, 0)),
            full(D, H), full(D, H), full(1, H), full(1, H), full(1, H),
            full(H, D), full(1, D),
        ],
        out_specs=pl.BlockSpec((BN, D), lambda i: (i, 0)),
        out_shape=jax.ShapeDtypeStruct((N, D), jnp.float32),
    )(h, p, wn1a, wn1b, bn1.reshape(1, H), gn.reshape(1, H),
      bbn.reshape(1, H), wn2, bn2.reshape(1, D))


def kernel(h, edge_index, edge_attr, W_e1, b_e1, g_e, bb_e, W_e2, b_e2,
           W_n1, b_n1, g_n, bb_n, W_n2, b_n2):
    N, D = h.shape
    row, col = edge_index[0], edge_index[1]
    S, T = _sc_gather(h, row, col)
    mij = _tc_edge(S, T, edge_attr, W_e1[:D], W_e1[D:2 * D], W_e1[2 * D:],
                   b_e1, g_e, bb_e, W_e2, b_e2)
    agg = _sc_scatter(mij, row, N)
    h_out = _tc_node(h, agg[:N], W_n1[:D], W_n1[D:], b_n1,
                     g_n, bb_n, W_n2, b_n2)
    return (h_out, mij)


# speedup vs baseline: 3.1150x; 1.0068x over previous
import functools

import jax
import jax.numpy as jnp
from jax import lax
from jax.experimental import pallas as pl
from jax.experimental.pallas import tpu as pltpu
from jax.experimental.pallas import tpu_sc as plsc

_NC = 2
_NS = 16
_NW = _NC * _NS


def _sc_gather(h, row, col):
    N, D = h.shape
    E = row.shape[0]
    per_w = E // _NW
    NBUF = 5
    C = next(cc for cc in (80, 40, 16, 8) if per_w % (cc * NBUF) == 0)
    n_ch = per_w // C
    rounds = n_ch // NBUF
    assert per_w * _NW == E and n_ch * C == per_w and rounds * NBUF == n_ch

    mesh = plsc.VectorSubcoreMesh(core_axis_name="c", subcore_axis_name="s",
                                  num_cores=_NC, num_subcores=_NS)

    @functools.partial(
        pl.kernel,
        out_type=(jax.ShapeDtypeStruct((E, D), jnp.float32),
                  jax.ShapeDtypeStruct((E, D), jnp.float32)),
        mesh=mesh,
        scratch_types=(
            [pltpu.VMEM((per_w,), jnp.int32), pltpu.VMEM((per_w,), jnp.int32)]
            + [pltpu.VMEM((C, D), jnp.float32)] * (2 * NBUF)
            + [pltpu.SemaphoreType.DMA] * (2 * NBUF)
        ),
    )
    def k(h_hbm, row_hbm, col_hbm, s_out, t_out, ridx, cidx, *rest):
        sbufs = rest[0:NBUF]
        tbufs = rest[NBUF:2 * NBUF]
        gsems = rest[2 * NBUF:3 * NBUF]
        wsems = rest[3 * NBUF:4 * NBUF]
        c = lax.axis_index("c")
        s = lax.axis_index("s")
        w = s * _NC + c
        base = w * per_w
        pltpu.sync_copy(row_hbm.at[pl.ds(base, per_w)], ridx)
        pltpu.sync_copy(col_hbm.at[pl.ds(base, per_w)], cidx)

        def fire(ch, b):
            pltpu.async_copy(h_hbm.at[ridx.at[pl.ds(ch * C, C)]], sbufs[b],
                             gsems[b])
            pltpu.async_copy(h_hbm.at[cidx.at[pl.ds(ch * C, C)]], tbufs[b],
                             gsems[b])

        for b in range(NBUF):
            fire(b, b)

        def round_(q, carry):
            for b in range(NBUF):
                ch = q * NBUF + b
                off = base + ch * C
                pltpu.make_async_copy(h_hbm.at[ridx.at[pl.ds(0, C)]],
                                      sbufs[b], gsems[b]).wait()
                pltpu.make_async_copy(h_hbm.at[cidx.at[pl.ds(0, C)]],
                                      tbufs[b], gsems[b]).wait()
                pltpu.async_copy(sbufs[b], s_out.at[pl.ds(off, C)], wsems[b])
                pltpu.async_copy(tbufs[b], t_out.at[pl.ds(off, C)], wsems[b])
            for b in range(NBUF):
                ch = q * NBUF + b
                pltpu.make_async_copy(sbufs[b], s_out.at[pl.ds(base, C)],
                                      wsems[b]).wait()
                pltpu.make_async_copy(tbufs[b], t_out.at[pl.ds(base, C)],
                                      wsems[b]).wait()

                @pl.when(q < rounds - 1)
                def _():
                    fire(ch + NBUF, b)

            return carry

        lax.fori_loop(0, rounds, round_, 0)

    return k(h, row, col)


def _sc_scatter(mij, row, N):
    E, H = mij.shape
    per_w = E // _NS
    C = 80
    n_ch = per_w // C
    NBUF = 5
    rounds = n_ch // NBUF
    rpt = -(-N // (8 * _NS * _NC)) * 8
    rpc = rpt * _NS
    Np = rpc * _NC
    lanes = C // 16
    assert n_ch * C == per_w and rounds * NBUF == n_ch and rpt % C == 0

    mesh = plsc.VectorSubcoreMesh(core_axis_name="c", subcore_axis_name="s",
                                  num_cores=_NC, num_subcores=_NS)

    @functools.partial(
        pl.kernel,
        out_type=jax.ShapeDtypeStruct((Np, H), jnp.float32),
        mesh=mesh,
        scratch_types=(
            [pltpu.VMEM((per_w,), jnp.int32)]
            + [pltpu.VMEM((C, H), jnp.float32)] * NBUF
            + [pltpu.VMEM((C,), jnp.int32)] * NBUF
            + [pltpu.VMEM_SHARED((rpc + 8, H), jnp.float32)]
            + [pltpu.SemaphoreType.DMA] * (2 * NBUF)
        ),
    )
    def k(mij_hbm, row_hbm, out_hbm, idx_all, *rest):
        bufs = rest[0:NBUF]
        idx2 = rest[NBUF:2 * NBUF]
        agg_sh = rest[2 * NBUF]
        lsems = rest[2 * NBUF + 1:3 * NBUF + 1]
        ssems = rest[3 * NBUF + 1:4 * NBUF + 1]
        c = lax.axis_index("c")
        s = lax.axis_index("s")
        hl = H // 16
        zv = jnp.zeros((16,), jnp.float32)

        def zrow(i, carry):
            bufs[0][i // hl, pl.ds((i % hl) * 16, 16)] = zv
            return carry

        lax.fori_loop(0, C * hl, zrow, 0)
        for t in range(rpt // C):
            pltpu.sync_copy(bufs[0], agg_sh.at[pl.ds(s * rpt + t * C, C)])

        @pl.when(s == 0)
        def _():
            pltpu.sync_copy(bufs[0].at[pl.ds(0, 8)], agg_sh.at[pl.ds(rpc, 8)])

        base_row = c * rpc
        base = s * per_w
        pltpu.sync_copy(row_hbm.at[pl.ds(base, per_w)], idx_all)
        plsc.subcore_barrier()

        def fire(ch, b):
            pltpu.async_copy(mij_hbm.at[pl.ds(base + ch * C, C)], bufs[b],
                             lsems[b])

        for b in range(NBUF):
            fire(b, b)

        def round_(q, carry):
            for b in range(NBUF):
                ch = q * NBUF + b
                pltpu.make_async_copy(mij_hbm.at[pl.ds(base, C)], bufs[b],
                                      lsems[b]).wait()
                for j in range(lanes):
                    v = idx_all[pl.ds(ch * C + j * 16, 16)] - base_row
                    ok = (v >= 0) & (v < rpc)
                    idx2[b][pl.ds(j * 16, 16)] = jnp.where(ok, v, rpc)
                pltpu.async_copy(bufs[b], agg_sh.at[idx2[b]], ssems[b],
                                 add=True)
            for b in range(NBUF):
                ch = q * NBUF + b
                pltpu.make_async_copy(bufs[b], agg_sh.at[idx2[b]],
                                      ssems[b]).wait()

                @pl.when(q < rounds - 1)
                def _():
                    fire(ch + NBUF, b)

            return carry

        lax.fori_loop(0, rounds, round_, 0)
        plsc.subcore_barrier()
        pltpu.sync_copy(agg_sh.at[pl.ds(s * rpt, rpt)],
                        out_hbm.at[pl.ds(base_row + s * rpt, rpt)])

    return k(mij, row)


def _silu(x):
    return x * jax.nn.sigmoid(x)


def _tc_edge(S, T, ea, w1a, w1b, w1c, b1, g, bb, w2, b2, Etot, base,
             mij_prev=None):
    E, D = S.shape
    DE = ea.shape[1]
    H = w2.shape[1]
    BE = 2560 if E % 2560 == 0 else (2000 if E % 2000 == 0 else E)
    grid = E // BE
    base_blk = base // BE
    assert grid * BE == E and base_blk * BE == base

    def body(s_ref, t_ref, e_ref, w1a_r, w1b_r, w1c_r, b1_r, g_r, bb_r,
             w2_r, b2_r, *rest):
        out_ref = rest[-1]
        x = (jnp.dot(s_ref[...], w1a_r[...], preferred_element_type=jnp.float32)
             + jnp.dot(t_ref[...], w1b_r[...], preferred_element_type=jnp.float32)
             + jnp.dot(e_ref[...], w1c_r[...], preferred_element_type=jnp.float32)
             + b1_r[...])
        mu = jnp.mean(x, axis=-1, keepdims=True)
        var = jnp.mean((x - mu) ** 2, axis=-1, keepdims=True)
        xn = (x - mu) / jnp.sqrt(var + 1e-5) * g_r[...] + bb_r[...]
        m = _silu(xn)
        y = jnp.dot(m, w2_r[...], preferred_element_type=jnp.float32) + b2_r[...]
        out_ref[...] = _silu(y)

    full = lambda r, c: pl.BlockSpec((r, c), lambda i: (0, 0))
    in_specs = [
        pl.BlockSpec((BE, D), lambda i: (i, 0)),
        pl.BlockSpec((BE, D), lambda i: (i, 0)),
        pl.BlockSpec((BE, DE), lambda i: (i, 0)),
        full(D, H), full(D, H), full(DE, H), full(1, H), full(1, H),
        full(1, H), full(H, H), full(1, H),
    ]
    args = [S, T, ea, w1a, w1b, w1c, b1.reshape(1, H), g.reshape(1, H),
            bb.reshape(1, H), w2, b2.reshape(1, H)]
    kwargs = {}
    if mij_prev is not None:
        in_specs.append(pl.BlockSpec(memory_space=pl.ANY))
        args.append(mij_prev)
        kwargs["input_output_aliases"] = {len(args) - 1: 0}
    return pl.pallas_call(
        body,
        grid=(grid,),
        in_specs=in_specs,
        out_specs=pl.BlockSpec((BE, H), lambda i: (i + base_blk, 0)),
        out_shape=jax.ShapeDtypeStruct((Etot, H), jnp.float32),
        **kwargs,
    )(*args)


def _tc_node(h, p, wn1a, wn1b, bn1, gn, bbn, wn2, bn2):
    N, D = h.shape
    H = wn1a.shape[1]
    BN = 2000 if N % 2000 == 0 else N
    grid = N // BN
    assert grid * BN == N

    def body(h_ref, p_ref, wa_r, wb_r, b1_r, g_r, bb_r, w2_r, b2_r,
             out_ref):
        agg = p_ref[...] * jnp.float32(0.01)
        x = (jnp.dot(h_ref[...], wa_r[...], preferred_element_type=jnp.float32)
             + jnp.dot(agg, wb_r[...], preferred_element_type=jnp.float32)
             + b1_r[...])
        mu = jnp.mean(x, axis=-1, keepdims=True)
        var = jnp.mean((x - mu) ** 2, axis=-1, keepdims=True)
        xn = (x - mu) / jnp.sqrt(var + 1e-5) * g_r[...] + bb_r[...]
        nh = _silu(xn)
        y = jnp.dot(nh, w2_r[...], preferred_element_type=jnp.float32) + b2_r[...]
        out_ref[...] = h_ref[...] + y

    full = lambda r, c: pl.BlockSpec((r, c), lambda i: (0, 0))
    return pl.pallas_call(
        body,
        grid=(grid,),
        in_specs=[
            pl.BlockSpec((BN, D), lambda i: (i, 0)),
            pl.BlockSpec((BN, D), lambda i: (i, 0)),
            full(D, H), full(D, H), full(1, H), full(1, H), full(1, H),
            full(H, D), full(1, D),
        ],
        out_specs=pl.BlockSpec((BN, D), lambda i: (i, 0)),
        out_shape=jax.ShapeDtypeStruct((N, D), jnp.float32),
    )(h, p, wn1a, wn1b, bn1.reshape(1, H), gn.reshape(1, H),
      bbn.reshape(1, H), wn2, bn2.reshape(1, D))


def kernel(h, edge_index, edge_attr, W_e1, b_e1, g_e, bb_e, W_e2, b_e2,
           W_n1, b_n1, g_n, bb_n, W_n2, b_n2):
    N, D = h.shape
    E = edge_index.shape[1]
    E2 = E // 2
    row, col = edge_index[0], edge_index[1]
    ew = (W_e1[:D], W_e1[D:2 * D], W_e1[2 * D:], b_e1, g_e, bb_e, W_e2, b_e2)
    SA, TA = _sc_gather(h, row[:E2], col[:E2])
    SB, TB = _sc_gather(h, row[E2:], col[E2:])
    mijA = _tc_edge(SA, TA, edge_attr[:E2], *ew, Etot=E, base=0)
    mij = _tc_edge(SB, TB, edge_attr[E2:], *ew, Etot=E, base=E2,
                   mij_prev=mijA)
    agg = _sc_scatter(mij, row, N)
    h_out = _tc_node(h, agg[:N], W_n1[:D], W_n1[D:], b_n1,
                     g_n, bb_n, W_n2, b_n2)
    return (h_out, mij)


# speedup vs baseline: 3.5199x; 1.1300x over previous
import functools

import jax
import jax.numpy as jnp
from jax import lax
from jax.experimental import pallas as pl
from jax.experimental.pallas import tpu as pltpu
from jax.experimental.pallas import tpu_sc as plsc

_NC = 2
_NS = 16
_NW = _NC * _NS


def _sc_gather(h, row, col):
    N, D = h.shape
    E = row.shape[0]
    per_w = E // _NW
    NBUF = 5
    C = next(cc for cc in (80, 40, 16, 8) if per_w % (cc * NBUF) == 0)
    n_ch = per_w // C
    rounds = n_ch // NBUF
    assert per_w * _NW == E and n_ch * C == per_w and rounds * NBUF == n_ch

    mesh = plsc.VectorSubcoreMesh(core_axis_name="c", subcore_axis_name="s",
                                  num_cores=_NC, num_subcores=_NS)

    @functools.partial(
        pl.kernel,
        out_type=(jax.ShapeDtypeStruct((E, D), jnp.float32),
                  jax.ShapeDtypeStruct((E, D), jnp.float32)),
        mesh=mesh,
        scratch_types=(
            [pltpu.VMEM((per_w,), jnp.int32), pltpu.VMEM((per_w,), jnp.int32)]
            + [pltpu.VMEM((C, D), jnp.float32)] * (2 * NBUF)
            + [pltpu.SemaphoreType.DMA] * (2 * NBUF)
        ),
    )
    def k(h_hbm, row_hbm, col_hbm, s_out, t_out, ridx, cidx, *rest):
        sbufs = rest[0:NBUF]
        tbufs = rest[NBUF:2 * NBUF]
        gsems = rest[2 * NBUF:3 * NBUF]
        wsems = rest[3 * NBUF:4 * NBUF]
        c = lax.axis_index("c")
        s = lax.axis_index("s")
        w = s * _NC + c
        base = w * per_w
        pltpu.sync_copy(row_hbm.at[pl.ds(base, per_w)], ridx)
        pltpu.sync_copy(col_hbm.at[pl.ds(base, per_w)], cidx)

        def fire(ch, b):
            pltpu.async_copy(h_hbm.at[ridx.at[pl.ds(ch * C, C)]], sbufs[b],
                             gsems[b])
            pltpu.async_copy(h_hbm.at[cidx.at[pl.ds(ch * C, C)]], tbufs[b],
                             gsems[b])

        for b in range(NBUF):
            fire(b, b)

        def round_(q, carry):
            for b in range(NBUF):
                ch = q * NBUF + b
                off = base + ch * C
                pltpu.make_async_copy(h_hbm.at[ridx.at[pl.ds(0, C)]],
                                      sbufs[b], gsems[b]).wait()
                pltpu.make_async_copy(h_hbm.at[cidx.at[pl.ds(0, C)]],
                                      tbufs[b], gsems[b]).wait()
                pltpu.async_copy(sbufs[b], s_out.at[pl.ds(off, C)], wsems[b])
                pltpu.async_copy(tbufs[b], t_out.at[pl.ds(off, C)], wsems[b])
            for b in range(NBUF):
                ch = q * NBUF + b
                pltpu.make_async_copy(sbufs[b], s_out.at[pl.ds(base, C)],
                                      wsems[b]).wait()
                pltpu.make_async_copy(tbufs[b], t_out.at[pl.ds(base, C)],
                                      wsems[b]).wait()

                @pl.when(q < rounds - 1)
                def _():
                    fire(ch + NBUF, b)

            return carry

        lax.fori_loop(0, rounds, round_, 0)

    return k(h, row, col)


def _sc_scatter(mij, row, N, base_e=0, count=None):
    E_all, H = mij.shape
    E = E_all - base_e if count is None else count
    per_w = E // _NS
    C = 80
    n_ch = per_w // C
    NBUF = 5
    rounds = n_ch // NBUF
    rpt = -(-N // (8 * _NS * _NC)) * 8
    rpc = rpt * _NS
    Np = rpc * _NC
    lanes = C // 16
    assert n_ch * C == per_w and rounds * NBUF == n_ch and rpt % C == 0

    mesh = plsc.VectorSubcoreMesh(core_axis_name="c", subcore_axis_name="s",
                                  num_cores=_NC, num_subcores=_NS)

    @functools.partial(
        pl.kernel,
        out_type=jax.ShapeDtypeStruct((Np, H), jnp.float32),
        mesh=mesh,
        scratch_types=(
            [pltpu.VMEM((per_w,), jnp.int32)]
            + [pltpu.VMEM((C, H), jnp.float32)] * NBUF
            + [pltpu.VMEM((C,), jnp.int32)] * NBUF
            + [pltpu.VMEM_SHARED((rpc + 8, H), jnp.float32)]
            + [pltpu.SemaphoreType.DMA] * (2 * NBUF)
        ),
    )
    def k(mij_hbm, row_hbm, out_hbm, idx_all, *rest):
        bufs = rest[0:NBUF]
        idx2 = rest[NBUF:2 * NBUF]
        agg_sh = rest[2 * NBUF]
        lsems = rest[2 * NBUF + 1:3 * NBUF + 1]
        ssems = rest[3 * NBUF + 1:4 * NBUF + 1]
        c = lax.axis_index("c")
        s = lax.axis_index("s")
        hl = H // 16
        zv = jnp.zeros((16,), jnp.float32)

        def zrow(i, carry):
            bufs[0][i // hl, pl.ds((i % hl) * 16, 16)] = zv
            return carry

        lax.fori_loop(0, C * hl, zrow, 0)
        for t in range(rpt // C):
            pltpu.sync_copy(bufs[0], agg_sh.at[pl.ds(s * rpt + t * C, C)])

        @pl.when(s == 0)
        def _():
            pltpu.sync_copy(bufs[0].at[pl.ds(0, 8)], agg_sh.at[pl.ds(rpc, 8)])

        base_row = c * rpc
        base = base_e + s * per_w
        pltpu.sync_copy(row_hbm.at[pl.ds(base, per_w)], idx_all)
        plsc.subcore_barrier()

        def fire(ch, b):
            pltpu.async_copy(mij_hbm.at[pl.ds(base + ch * C, C)], bufs[b],
                             lsems[b])

        for b in range(NBUF):
            fire(b, b)

        def round_(q, carry):
            for b in range(NBUF):
                ch = q * NBUF + b
                pltpu.make_async_copy(mij_hbm.at[pl.ds(base, C)], bufs[b],
                                      lsems[b]).wait()
                for j in range(lanes):
                    v = idx_all[pl.ds(ch * C + j * 16, 16)] - base_row
                    ok = (v >= 0) & (v < rpc)
                    idx2[b][pl.ds(j * 16, 16)] = jnp.where(ok, v, rpc)
                pltpu.async_copy(bufs[b], agg_sh.at[idx2[b]], ssems[b],
                                 add=True)
            for b in range(NBUF):
                ch = q * NBUF + b
                pltpu.make_async_copy(bufs[b], agg_sh.at[idx2[b]],
                                      ssems[b]).wait()

                @pl.when(q < rounds - 1)
                def _():
                    fire(ch + NBUF, b)

            return carry

        lax.fori_loop(0, rounds, round_, 0)
        plsc.subcore_barrier()
        pltpu.sync_copy(agg_sh.at[pl.ds(s * rpt, rpt)],
                        out_hbm.at[pl.ds(base_row + s * rpt, rpt)])

    return k(mij, row)


def _silu(x):
    return x * jax.nn.sigmoid(x)


def _tc_edge(S, T, ea, w1a, w1b, w1c, b1, g, bb, w2, b2, Etot, base,
             mij_prev=None):
    E, D = S.shape
    DE = ea.shape[1]
    H = w2.shape[1]
    BE = 2560 if E % 2560 == 0 else (2000 if E % 2000 == 0 else E)
    grid = E // BE
    base_blk = base // BE
    assert grid * BE == E and base_blk * BE == base
    dup = mij_prev is None

    def body(s_ref, t_ref, e_ref, w1a_r, w1b_r, w1c_r, b1_r, g_r, bb_r,
             w2_r, b2_r, *rest):
        x = (jnp.dot(s_ref[...], w1a_r[...], preferred_element_type=jnp.float32)
             + jnp.dot(t_ref[...], w1b_r[...], preferred_element_type=jnp.float32)
             + jnp.dot(e_ref[...], w1c_r[...], preferred_element_type=jnp.float32)
             + b1_r[...])
        mu = jnp.mean(x, axis=-1, keepdims=True)
        var = jnp.mean((x - mu) ** 2, axis=-1, keepdims=True)
        xn = (x - mu) / jnp.sqrt(var + 1e-5) * g_r[...] + bb_r[...]
        m = _silu(xn)
        y = jnp.dot(m, w2_r[...], preferred_element_type=jnp.float32) + b2_r[...]
        val = _silu(y)
        if dup:
            rest[-2][...] = val
            rest[-1][...] = val
        else:
            rest[-1][...] = val

    full = lambda r, c: pl.BlockSpec((r, c), lambda i: (0, 0))
    in_specs = [
        pl.BlockSpec((BE, D), lambda i: (i, 0)),
        pl.BlockSpec((BE, D), lambda i: (i, 0)),
        pl.BlockSpec((BE, DE), lambda i: (i + base_blk, 0)),
        full(D, H), full(D, H), full(DE, H), full(1, H), full(1, H),
        full(1, H), full(H, H), full(1, H),
    ]
    args = [S, T, ea, w1a, w1b, w1c, b1.reshape(1, H), g.reshape(1, H),
            bb.reshape(1, H), w2, b2.reshape(1, H)]
    kwargs = {}
    out_specs = [pl.BlockSpec((BE, H), lambda i: (i + base_blk, 0))]
    out_shape = [jax.ShapeDtypeStruct((Etot, H), jnp.float32)]
    if dup:
        out_specs.append(pl.BlockSpec((BE, H), lambda i: (i, 0)))
        out_shape.append(jax.ShapeDtypeStruct((E, H), jnp.float32))
    else:
        in_specs.append(pl.BlockSpec(memory_space=pl.ANY))
        args.append(mij_prev)
        kwargs["input_output_aliases"] = {len(args) - 1: 0}
    res = pl.pallas_call(
        body,
        grid=(grid,),
        in_specs=in_specs,
        out_specs=out_specs,
        out_shape=out_shape,
        **kwargs,
    )(*args)
    return res if dup else res[0]


def _tc_node(h, p, q, wn1a, wn1b, bn1, gn, bbn, wn2, bn2):
    N, D = h.shape
    H = wn1a.shape[1]
    BN = 2000 if N % 2000 == 0 else N
    grid = N // BN
    assert grid * BN == N

    def body(h_ref, p_ref, q_ref, wa_r, wb_r, b1_r, g_r, bb_r, w2_r, b2_r,
             out_ref):
        agg = (p_ref[...] + q_ref[...]) * jnp.float32(0.01)
        x = (jnp.dot(h_ref[...], wa_r[...], preferred_element_type=jnp.float32)
             + jnp.dot(agg, wb_r[...], preferred_element_type=jnp.float32)
             + b1_r[...])
        mu = jnp.mean(x, axis=-1, keepdims=True)
        var = jnp.mean((x - mu) ** 2, axis=-1, keepdims=True)
        xn = (x - mu) / jnp.sqrt(var + 1e-5) * g_r[...] + bb_r[...]
        nh = _silu(xn)
        y = jnp.dot(nh, w2_r[...], preferred_element_type=jnp.float32) + b2_r[...]
        out_ref[...] = h_ref[...] + y

    full = lambda r, c: pl.BlockSpec((r, c), lambda i: (0, 0))
    return pl.pallas_call(
        body,
        grid=(grid,),
        in_specs=[
            pl.BlockSpec((BN, D), lambda i: (i, 0)),
            pl.BlockSpec((BN, D), lambda i: (i, 0)),
            pl.BlockSpec((BN, D), lambda i: (i, 0)),
            full(D, H), full(D, H), full(1, H), full(1, H), full(1, H),
            full(H, D), full(1, D),
        ],
        out_specs=pl.BlockSpec((BN, D), lambda i: (i, 0)),
        out_shape=jax.ShapeDtypeStruct((N, D), jnp.float32),
    )(h, p, q, wn1a, wn1b, bn1.reshape(1, H), gn.reshape(1, H),
      bbn.reshape(1, H), wn2, bn2.reshape(1, D))


def kernel(h, edge_index, edge_attr, W_e1, b_e1, g_e, bb_e, W_e2, b_e2,
           W_n1, b_n1, g_n, bb_n, W_n2, b_n2):
    N, D = h.shape
    E = edge_index.shape[1]
    E2 = E // 2
    row, col = edge_index[0], edge_index[1]
    ew = (W_e1[:D], W_e1[D:2 * D], W_e1[2 * D:], b_e1, g_e, bb_e, W_e2, b_e2)
    SA, TA = _sc_gather(h, row[:E2], col[:E2])
    SB, TB = _sc_gather(h, row[E2:], col[E2:])
    mijA, mijA_dup = _tc_edge(SA, TA, edge_attr, *ew, Etot=E, base=0)
    aggA = _sc_scatter(mijA_dup, row, N)
    mij = _tc_edge(SB, TB, edge_attr, *ew, Etot=E, base=E2,
                   mij_prev=mijA)
    aggB = _sc_scatter(mij, row, N, base_e=E2)
    h_out = _tc_node(h, aggA[:N], aggB[:N], W_n1[:D], W_n1[D:], b_n1,
                     g_n, bb_n, W_n2, b_n2)
    return (h_out, mij)


# speedup vs baseline: 3.6519x; 1.0375x over previous
import functools

import jax
import jax.numpy as jnp
from jax import lax
from jax.experimental import pallas as pl
from jax.experimental.pallas import tpu as pltpu
from jax.experimental.pallas import tpu_sc as plsc

_NC = 2
_NS = 16
_NW = _NC * _NS


def _sc_gather(h, row, col):
    N, D = h.shape
    E = row.shape[0]
    per_w = E // _NW
    NBUF = 5
    C = next(cc for cc in (80, 40, 16, 8) if per_w % (cc * NBUF) == 0)
    n_ch = per_w // C
    rounds = n_ch // NBUF
    assert per_w * _NW == E and n_ch * C == per_w and rounds * NBUF == n_ch

    mesh = plsc.VectorSubcoreMesh(core_axis_name="c", subcore_axis_name="s",
                                  num_cores=_NC, num_subcores=_NS)

    @functools.partial(
        pl.kernel,
        out_type=(jax.ShapeDtypeStruct((E, D), jnp.float32),
                  jax.ShapeDtypeStruct((E, D), jnp.float32)),
        mesh=mesh,
        scratch_types=(
            [pltpu.VMEM((per_w,), jnp.int32), pltpu.VMEM((per_w,), jnp.int32)]
            + [pltpu.VMEM((C, D), jnp.float32)] * (2 * NBUF)
            + [pltpu.SemaphoreType.DMA] * (2 * NBUF)
        ),
    )
    def k(h_hbm, row_hbm, col_hbm, s_out, t_out, ridx, cidx, *rest):
        sbufs = rest[0:NBUF]
        tbufs = rest[NBUF:2 * NBUF]
        gsems = rest[2 * NBUF:3 * NBUF]
        wsems = rest[3 * NBUF:4 * NBUF]
        c = lax.axis_index("c")
        s = lax.axis_index("s")
        w = s * _NC + c
        base = w * per_w
        pltpu.sync_copy(row_hbm.at[pl.ds(base, per_w)], ridx)
        pltpu.sync_copy(col_hbm.at[pl.ds(base, per_w)], cidx)

        def fire(ch, b):
            pltpu.async_copy(h_hbm.at[ridx.at[pl.ds(ch * C, C)]], sbufs[b],
                             gsems[b])
            pltpu.async_copy(h_hbm.at[cidx.at[pl.ds(ch * C, C)]], tbufs[b],
                             gsems[b])

        for b in range(NBUF):
            fire(b, b)

        def round_(q, carry):
            for b in range(NBUF):
                ch = q * NBUF + b
                off = base + ch * C
                pltpu.make_async_copy(h_hbm.at[ridx.at[pl.ds(0, C)]],
                                      sbufs[b], gsems[b]).wait()
                pltpu.make_async_copy(h_hbm.at[cidx.at[pl.ds(0, C)]],
                                      tbufs[b], gsems[b]).wait()
                pltpu.async_copy(sbufs[b], s_out.at[pl.ds(off, C)], wsems[b])
                pltpu.async_copy(tbufs[b], t_out.at[pl.ds(off, C)], wsems[b])
            for b in range(NBUF):
                ch = q * NBUF + b
                pltpu.make_async_copy(sbufs[b], s_out.at[pl.ds(base, C)],
                                      wsems[b]).wait()
                pltpu.make_async_copy(tbufs[b], t_out.at[pl.ds(base, C)],
                                      wsems[b]).wait()

                @pl.when(q < rounds - 1)
                def _():
                    fire(ch + NBUF, b)

            return carry

        lax.fori_loop(0, rounds, round_, 0)

    return k(h, row, col)


def _sc_scatter(mij, row, N, base_e=0, row_base=None, count=None):
    E_all, H = mij.shape
    E = E_all - base_e if count is None else count
    if row_base is None:
        row_base = base_e
    per_w = E // _NS
    NBUF = 5
    C = next(cc for cc in (80, 40, 20, 16, 8) if per_w % (cc * NBUF) == 0)
    n_ch = per_w // C
    rounds = n_ch // NBUF
    rpt = -(-N // (8 * _NS * _NC)) * 8
    rpc = rpt * _NS
    Np = rpc * _NC
    lane_offs = list(range(0, C - 15, 16))
    if C % 16:
        lane_offs.append(C - 16)
    assert n_ch * C == per_w and rounds * NBUF == n_ch and rpt % C == 0

    mesh = plsc.VectorSubcoreMesh(core_axis_name="c", subcore_axis_name="s",
                                  num_cores=_NC, num_subcores=_NS)

    @functools.partial(
        pl.kernel,
        out_type=jax.ShapeDtypeStruct((Np, H), jnp.float32),
        mesh=mesh,
        scratch_types=(
            [pltpu.VMEM((per_w,), jnp.int32)]
            + [pltpu.VMEM((C, H), jnp.float32)] * NBUF
            + [pltpu.VMEM((C,), jnp.int32)] * NBUF
            + [pltpu.VMEM_SHARED((rpc + 8, H), jnp.float32)]
            + [pltpu.SemaphoreType.DMA] * (2 * NBUF)
        ),
    )
    def k(mij_hbm, row_hbm, out_hbm, idx_all, *rest):
        bufs = rest[0:NBUF]
        idx2 = rest[NBUF:2 * NBUF]
        agg_sh = rest[2 * NBUF]
        lsems = rest[2 * NBUF + 1:3 * NBUF + 1]
        ssems = rest[3 * NBUF + 1:4 * NBUF + 1]
        c = lax.axis_index("c")
        s = lax.axis_index("s")
        hl = H // 16
        zv = jnp.zeros((16,), jnp.float32)

        def zrow(i, carry):
            bufs[0][i // hl, pl.ds((i % hl) * 16, 16)] = zv
            return carry

        lax.fori_loop(0, C * hl, zrow, 0)
        for t in range(rpt // C):
            pltpu.sync_copy(bufs[0], agg_sh.at[pl.ds(s * rpt + t * C, C)])

        @pl.when(s == 0)
        def _():
            pltpu.sync_copy(bufs[0].at[pl.ds(0, 8)], agg_sh.at[pl.ds(rpc, 8)])

        base_row = c * rpc
        base = base_e + s * per_w
        pltpu.sync_copy(row_hbm.at[pl.ds(row_base + s * per_w, per_w)],
                        idx_all)
        plsc.subcore_barrier()

        def fire(ch, b):
            pltpu.async_copy(mij_hbm.at[pl.ds(base + ch * C, C)], bufs[b],
                             lsems[b])

        for b in range(NBUF):
            fire(b, b)

        def round_(q, carry):
            for b in range(NBUF):
                ch = q * NBUF + b
                pltpu.make_async_copy(mij_hbm.at[pl.ds(base, C)], bufs[b],
                                      lsems[b]).wait()
                for o in lane_offs:
                    v = idx_all[pl.ds(ch * C + o, 16)] - base_row
                    ok = (v >= 0) & (v < rpc)
                    idx2[b][pl.ds(o, 16)] = jnp.where(ok, v, rpc)
                pltpu.async_copy(bufs[b], agg_sh.at[idx2[b]], ssems[b],
                                 add=True)
            for b in range(NBUF):
                ch = q * NBUF + b
                pltpu.make_async_copy(bufs[b], agg_sh.at[idx2[b]],
                                      ssems[b]).wait()

                @pl.when(q < rounds - 1)
                def _():
                    fire(ch + NBUF, b)

            return carry

        lax.fori_loop(0, rounds, round_, 0)
        plsc.subcore_barrier()
        pltpu.sync_copy(agg_sh.at[pl.ds(s * rpt, rpt)],
                        out_hbm.at[pl.ds(base_row + s * rpt, rpt)])

    return k(mij, row)


def _silu(x):
    return x * jax.nn.sigmoid(x)


def _tc_edge(S, T, ea, w1a, w1b, w1c, b1, g, bb, w2, b2, Etot, base,
             mij_prev=None, want_dup=True, s_base=0, count=None):
    Es, D = S.shape
    E = Es if count is None else count
    DE = ea.shape[1]
    H = w2.shape[1]
    BE = 2560 if E % 2560 == 0 else (2000 if E % 2000 == 0 else E)
    grid = E // BE
    base_blk = base // BE
    s_blk = s_base // BE
    assert grid * BE == E and base_blk * BE == base and s_blk * BE == s_base
    dup = want_dup

    def body(s_ref, t_ref, e_ref, w1a_r, w1b_r, w1c_r, b1_r, g_r, bb_r,
             w2_r, b2_r, *rest):
        x = (jnp.dot(s_ref[...], w1a_r[...], preferred_element_type=jnp.float32)
             + jnp.dot(t_ref[...], w1b_r[...], preferred_element_type=jnp.float32)
             + jnp.dot(e_ref[...], w1c_r[...], preferred_element_type=jnp.float32)
             + b1_r[...])
        mu = jnp.mean(x, axis=-1, keepdims=True)
        var = jnp.mean((x - mu) ** 2, axis=-1, keepdims=True)
        xn = (x - mu) / jnp.sqrt(var + 1e-5) * g_r[...] + bb_r[...]
        m = _silu(xn)
        y = jnp.dot(m, w2_r[...], preferred_element_type=jnp.float32) + b2_r[...]
        val = _silu(y)
        if dup:
            rest[-2][...] = val
            rest[-1][...] = val
        else:
            rest[-1][...] = val

    full = lambda r, c: pl.BlockSpec((r, c), lambda i: (0, 0))
    in_specs = [
        pl.BlockSpec((BE, D), lambda i: (i + s_blk, 0)),
        pl.BlockSpec((BE, D), lambda i: (i + s_blk, 0)),
        pl.BlockSpec((BE, DE), lambda i: (i + base_blk, 0)),
        full(D, H), full(D, H), full(DE, H), full(1, H), full(1, H),
        full(1, H), full(H, H), full(1, H),
    ]
    args = [S, T, ea, w1a, w1b, w1c, b1.reshape(1, H), g.reshape(1, H),
            bb.reshape(1, H), w2, b2.reshape(1, H)]
    kwargs = {}
    out_specs = [pl.BlockSpec((BE, H), lambda i: (i + base_blk, 0))]
    out_shape = [jax.ShapeDtypeStruct((Etot, H), jnp.float32)]
    if dup:
        out_specs.append(pl.BlockSpec((BE, H), lambda i: (i, 0)))
        out_shape.append(jax.ShapeDtypeStruct((E, H), jnp.float32))
    if mij_prev is not None:
        in_specs.append(pl.BlockSpec(memory_space=pl.ANY))
        args.append(mij_prev)
        kwargs["input_output_aliases"] = {len(args) - 1: 0}
    res = pl.pallas_call(
        body,
        grid=(grid,),
        in_specs=in_specs,
        out_specs=out_specs,
        out_shape=out_shape,
        **kwargs,
    )(*args)
    return res if dup else res[0]


def _tc_node(h, aggs, wn1a, wn1b, bn1, gn, bbn, wn2, bn2):
    n_agg = len(aggs)
    N, D = h.shape
    H = wn1a.shape[1]
    BN = 2000 if N % 2000 == 0 else N
    grid = N // BN
    assert grid * BN == N

    def body(h_ref, *rest):
        (wa_r, wb_r, b1_r, g_r, bb_r, w2_r, b2_r, out_ref) = rest[n_agg:]
        acc = rest[0][...]
        for r in rest[1:n_agg]:
            acc = acc + r[...]
        agg = acc * jnp.float32(0.01)
        x = (jnp.dot(h_ref[...], wa_r[...], preferred_element_type=jnp.float32)
             + jnp.dot(agg, wb_r[...], preferred_element_type=jnp.float32)
             + b1_r[...])
        mu = jnp.mean(x, axis=-1, keepdims=True)
        var = jnp.mean((x - mu) ** 2, axis=-1, keepdims=True)
        xn = (x - mu) / jnp.sqrt(var + 1e-5) * g_r[...] + bb_r[...]
        nh = _silu(xn)
        y = jnp.dot(nh, w2_r[...], preferred_element_type=jnp.float32) + b2_r[...]
        out_ref[...] = h_ref[...] + y

    full = lambda r, c: pl.BlockSpec((r, c), lambda i: (0, 0))
    return pl.pallas_call(
        body,
        grid=(grid,),
        in_specs=(
            [pl.BlockSpec((BN, D), lambda i: (i, 0))] * (1 + n_agg)
            + [full(D, H), full(D, H), full(1, H), full(1, H), full(1, H),
               full(H, D), full(1, D)]
        ),
        out_specs=pl.BlockSpec((BN, D), lambda i: (i, 0)),
        out_shape=jax.ShapeDtypeStruct((N, D), jnp.float32),
    )(h, *aggs, wn1a, wn1b, bn1.reshape(1, H), gn.reshape(1, H),
      bbn.reshape(1, H), wn2, bn2.reshape(1, D))


def kernel(h, edge_index, edge_attr, W_e1, b_e1, g_e, bb_e, W_e2, b_e2,
           W_n1, b_n1, g_n, bb_n, W_n2, b_n2):
    N, D = h.shape
    E = edge_index.shape[1]
    K = 4
    Ek = E // K
    row, col = edge_index[0], edge_index[1]
    ew = (W_e1[:D], W_e1[D:2 * D], W_e1[2 * D:], b_e1, g_e, bb_e, W_e2, b_e2)
    E2 = E // 2
    SA, TA = _sc_gather(h, row[:E2], col[:E2])
    SB, TB = _sc_gather(h, row[E2:], col[E2:])
    aggs = []
    mij = None
    for k in range(K):
        Sk, Tk = (SA, TA) if k < K // 2 else (SB, TB)
        last = k == K - 1
        res = _tc_edge(Sk, Tk, edge_attr, *ew, Etot=E, base=k * Ek,
                       mij_prev=mij, want_dup=not last,
                       s_base=(k % (K // 2)) * Ek, count=Ek)
        if last:
            mij = res
            aggs.append(_sc_scatter(mij, row, N, base_e=k * Ek))
        else:
            mij, dup = res
            aggs.append(_sc_scatter(dup, row, N, row_base=k * Ek))
    h_out = _tc_node(h, [a[:N] for a in aggs], W_n1[:D], W_n1[D:], b_n1,
                     g_n, bb_n, W_n2, b_n2)
    return (h_out, mij)
